# initial kernel scaffold (unmeasured)
import jax
import jax.numpy as jnp
from jax import lax
from jax.experimental import pallas as pl
from jax.experimental.pallas import tpu as pltpu

N_DEV = 32
M = 4096
N_COL = 2048
CH = M // N_DEV
N_HOPS = 2 * N_DEV - 2


def kernel(x, w_mat, scale_x, scale_w):
    xb = x.astype(jnp.bfloat16)
    wb = w_mat.astype(jnp.bfloat16)
    sp = (scale_x.astype(jnp.float32) * scale_w.astype(jnp.float32)).reshape(1, 1)

    def body(x_ref, w_ref, sp_ref, out_ref, stage, comm,
             send_sems, recv_sems, credit_sem, copy_sem):
        d = lax.axis_index("i")
        left = lax.rem(d - 1 + N_DEV, N_DEV)
        right = lax.rem(d + 1, N_DEV)

        barrier = pltpu.get_barrier_semaphore()
        for nbr in (left, right):
            pl.semaphore_signal(barrier, inc=1, device_id=(nbr,),
                                device_id_type=pl.DeviceIdType.MESH)
        pl.semaphore_wait(barrier, 2)

        scale = sp_ref[0, 0]

        def pchunk(idx):
            xa = x_ref[pl.ds(idx * CH, CH), :]
            return lax.dot_general(
                xa, w_ref[...],
                dimension_numbers=(((1,), (0,)), ((), ())),
                preferred_element_type=jnp.float32)

        stage[0, :, :] = pchunk(d)

        for h in range(N_HOPS):
            slot = h % 2
            nxt = (h + 1) % 2
            rdma = pltpu.make_async_remote_copy(
                src_ref=stage.at[slot],
                dst_ref=comm.at[slot],
                send_sem=send_sems.at[slot],
                recv_sem=recv_sems.at[slot],
                device_id=(right,),
                device_id_type=pl.DeviceIdType.MESH,
            )
            if h >= 2:
                pl.semaphore_wait(credit_sem, 1)
            rdma.start()
            rdma.wait_send()
            rdma.wait_recv()

            if h < N_DEV - 1:
                idx = lax.rem(d - (h + 1) + 2 * N_DEV, N_DEV)
                acc = comm[slot, :, :] + pchunk(idx)
                if h == N_DEV - 2:
                    y = acc * scale
                    y = y / (1.0 + jnp.exp(-jnp.clip(y, -60.0, 60.0)))
                    stage[nxt, :, :] = y
                    cp = pltpu.make_async_copy(
                        stage.at[nxt], out_ref.at[pl.ds(right * CH, CH), :],
                        copy_sem)
                    cp.start()
                    cp.wait()
                else:
                    stage[nxt, :, :] = acc
            else:
                t = h - (N_DEV - 1)
                idx = lax.rem(d - t + 2 * N_DEV, N_DEV)
                if h < N_HOPS - 1:
                    stage[nxt, :, :] = comm[slot, :, :]
                cp = pltpu.make_async_copy(
                    comm.at[slot], out_ref.at[pl.ds(idx * CH, CH), :],
                    copy_sem)
                cp.start()
                cp.wait()

            if h <= N_HOPS - 3:
                pl.semaphore_signal(credit_sem, inc=1, device_id=(left,),
                                    device_id_type=pl.DeviceIdType.MESH)

    return pl.pallas_call(
        body,
        out_shape=jax.ShapeDtypeStruct((M, N_COL), jnp.float32),
        in_specs=[
            pl.BlockSpec(memory_space=pltpu.VMEM),
            pl.BlockSpec(memory_space=pltpu.VMEM),
            pl.BlockSpec(memory_space=pltpu.SMEM),
        ],
        out_specs=pl.BlockSpec(memory_space=pltpu.ANY),
        scratch_shapes=[
            pltpu.VMEM((2, CH, N_COL), jnp.float32),
            pltpu.VMEM((2, CH, N_COL), jnp.float32),
            pltpu.SemaphoreType.DMA((2,)),
            pltpu.SemaphoreType.DMA((2,)),
            pltpu.SemaphoreType.REGULAR,
            pltpu.SemaphoreType.DMA,
        ],
        compiler_params=pltpu.CompilerParams(collective_id=0),
    )(xb, wb, sp)


# baseline (device time: 883219 ns/iter reference)
import jax
import jax.numpy as jnp
from jax import lax
from jax.experimental import pallas as pl
from jax.experimental.pallas import tpu as pltpu

N_DEV = 32
M = 4096
N_COL = 2048
CH = M // N_DEV
N_HOPS = 2 * N_DEV - 2


def kernel(x, w_mat, scale_x, scale_w):
    xb = x.astype(jnp.bfloat16)
    wb = w_mat.astype(jnp.bfloat16)
    sp = (scale_x.astype(jnp.float32) * scale_w.astype(jnp.float32)).reshape(1, 1)

    def body(x_ref, w_ref, sp_ref, out_ref, stage, comm,
             send_sems, recv_sems, credit_sem, copy_sem):
        d = lax.axis_index("i")
        left = lax.rem(d - 1 + N_DEV, N_DEV)
        right = lax.rem(d + 1, N_DEV)

        barrier = pltpu.get_barrier_semaphore()
        for nbr in (left, right):
            pl.semaphore_signal(barrier, inc=1, device_id=(nbr,),
                                device_id_type=pl.DeviceIdType.MESH)
        pl.semaphore_wait(barrier, 2)

        scale = sp_ref[0, 0]

        def pchunk(idx):
            xa = x_ref[pl.ds(idx * CH, CH), :]
            return lax.dot_general(
                xa, w_ref[...],
                dimension_numbers=(((1,), (0,)), ((), ())),
                preferred_element_type=jnp.float32)

        stage[0, :, :] = pchunk(d)

        for h in range(N_HOPS):
            slot = h % 2
            nxt = (h + 1) % 2
            rdma = pltpu.make_async_remote_copy(
                src_ref=stage.at[slot],
                dst_ref=comm.at[slot],
                send_sem=send_sems.at[slot],
                recv_sem=recv_sems.at[slot],
                device_id=(right,),
                device_id_type=pl.DeviceIdType.MESH,
            )
            if h >= 2:
                pl.semaphore_wait(credit_sem, 1)
            rdma.start()
            rdma.wait_send()
            rdma.wait_recv()

            if h < N_DEV - 1:
                idx = lax.rem(d - (h + 1) + 2 * N_DEV, N_DEV)
                acc = comm[slot, :, :] + pchunk(idx)
                if h == N_DEV - 2:
                    y = acc * scale
                    y = y / (1.0 + jnp.exp(-jnp.clip(y, -60.0, 60.0)))
                    stage[nxt, :, :] = y
                    cp = pltpu.make_async_copy(
                        stage.at[nxt], out_ref.at[pl.ds(right * CH, CH), :],
                        copy_sem)
                    cp.start()
                    cp.wait()
                else:
                    stage[nxt, :, :] = acc
            else:
                t = h - (N_DEV - 1)
                idx = lax.rem(d - t + 2 * N_DEV, N_DEV)
                if h < N_HOPS - 1:
                    stage[nxt, :, :] = comm[slot, :, :]
                cp = pltpu.make_async_copy(
                    comm.at[slot], out_ref.at[pl.ds(idx * CH, CH), :],
                    copy_sem)
                cp.start()
                cp.wait()

            if h <= N_HOPS - 3:
                pl.semaphore_signal(credit_sem, inc=1, device_id=(left,),
                                    device_id_type=pl.DeviceIdType.MESH)

    return pl.pallas_call(
        body,
        out_shape=jax.ShapeDtypeStruct((M, N_COL), jnp.float32),
        in_specs=[
            pl.BlockSpec(memory_space=pltpu.VMEM),
            pl.BlockSpec(memory_space=pltpu.VMEM),
            pl.BlockSpec(memory_space=pltpu.SMEM),
        ],
        out_specs=pl.BlockSpec(memory_space=pl.ANY),
        scratch_shapes=[
            pltpu.VMEM((2, CH, N_COL), jnp.float32),
            pltpu.VMEM((2, CH, N_COL), jnp.float32),
            pltpu.SemaphoreType.DMA((2,)),
            pltpu.SemaphoreType.DMA((2,)),
            pltpu.SemaphoreType.REGULAR,
            pltpu.SemaphoreType.DMA,
        ],
        compiler_params=pltpu.CompilerParams(collective_id=0),
    )(xb, wb, sp)


# device time: 489229 ns/iter; 1.8053x vs baseline; 1.8053x over previous
import jax
import jax.numpy as jnp
from jax import lax
from jax.experimental import pallas as pl
from jax.experimental.pallas import tpu as pltpu

N_DEV = 32
M = 4096
N_COL = 2048
HC = N_COL // 2
CH = M // N_DEV
N_HOPS = 2 * N_DEV - 2


def kernel(x, w_mat, scale_x, scale_w):
    xb = x.astype(jnp.bfloat16)
    wb = w_mat.astype(jnp.bfloat16)
    sp = (scale_x.astype(jnp.float32) * scale_w.astype(jnp.float32)).reshape(1, 1)

    def body(x_ref, w_ref, sp_ref, out_ref,
             stageR, stageL, commR, commL,
             sendR, sendL, recvR, recvL, creditR, creditL):
        d = lax.axis_index("i")
        left = lax.rem(d - 1 + N_DEV, N_DEV)
        right = lax.rem(d + 1, N_DEV)

        barrier = pltpu.get_barrier_semaphore()
        for nbr in (left, right):
            pl.semaphore_signal(barrier, inc=1, device_id=(nbr,),
                                device_id_type=pl.DeviceIdType.MESH)
        pl.semaphore_wait(barrier, 2)

        scale = sp_ref[0, 0]

        def pchunk(idx, dir_):
            xa = x_ref[pl.ds(idx * CH, CH), :]
            wa = w_ref[:, dir_ * HC:(dir_ + 1) * HC]
            return lax.dot_general(
                xa, wa,
                dimension_numbers=(((1,), (0,)), ((), ())),
                preferred_element_type=jnp.float32)

        def silu(acc):
            y = acc * scale
            return y / (1.0 + jnp.exp(-jnp.clip(y, -60.0, 60.0)))

        stageR[0, :, :] = pchunk(d, 0).astype(jnp.bfloat16)
        stageL[0, :, :] = pchunk(d, 1).astype(jnp.bfloat16)

        for h in range(N_HOPS):
            slot = h % 2
            nxt = (h + 1) % 2
            if h >= 2:
                pl.semaphore_wait(creditR, 1)
                pl.semaphore_wait(creditL, 1)
            rdma_r = pltpu.make_async_remote_copy(
                src_ref=stageR.at[slot], dst_ref=commR.at[slot],
                send_sem=sendR.at[slot], recv_sem=recvR.at[slot],
                device_id=(right,), device_id_type=pl.DeviceIdType.MESH)
            rdma_l = pltpu.make_async_remote_copy(
                src_ref=stageL.at[slot], dst_ref=commL.at[slot],
                send_sem=sendL.at[slot], recv_sem=recvL.at[slot],
                device_id=(left,), device_id_type=pl.DeviceIdType.MESH)
            rdma_r.start()
            rdma_l.start()
            rdma_r.wait_send()
            rdma_l.wait_send()
            rdma_r.wait_recv()
            rdma_l.wait_recv()

            for dir_ in (0, 1):
                comm = commR if dir_ == 0 else commL
                stage = stageR if dir_ == 0 else stageL
                cs = slice(dir_ * HC, (dir_ + 1) * HC)
                if h < N_DEV - 1:
                    if dir_ == 0:
                        idx = lax.rem(d - (h + 1) + 2 * N_DEV, N_DEV)
                    else:
                        idx = lax.rem(d + (h + 1), N_DEV)
                    acc = comm[slot, :, :].astype(jnp.float32) + pchunk(idx, dir_)
                    if h == N_DEV - 2:
                        y = silu(acc)
                        stage[nxt, :, :] = y.astype(jnp.bfloat16)
                        out_ref[pl.ds(idx * CH, CH), cs] = y
                    else:
                        stage[nxt, :, :] = acc.astype(jnp.bfloat16)
                else:
                    t = h - (N_DEV - 1)
                    if dir_ == 0:
                        idx = lax.rem(d - t + 2 * N_DEV, N_DEV)
                    else:
                        idx = lax.rem(d + t, N_DEV)
                    if h < N_HOPS - 1:
                        stage[nxt, :, :] = comm[slot, :, :]
                    out_ref[pl.ds(idx * CH, CH), cs] = (
                        comm[slot, :, :].astype(jnp.float32))

            if h <= N_HOPS - 3:
                pl.semaphore_signal(creditR, inc=1, device_id=(left,),
                                    device_id_type=pl.DeviceIdType.MESH)
                pl.semaphore_signal(creditL, inc=1, device_id=(right,),
                                    device_id_type=pl.DeviceIdType.MESH)

    return pl.pallas_call(
        body,
        out_shape=jax.ShapeDtypeStruct((M, N_COL), jnp.float32),
        in_specs=[
            pl.BlockSpec(memory_space=pltpu.VMEM),
            pl.BlockSpec(memory_space=pltpu.VMEM),
            pl.BlockSpec(memory_space=pltpu.SMEM),
        ],
        out_specs=pl.BlockSpec(memory_space=pltpu.VMEM),
        scratch_shapes=[
            pltpu.VMEM((2, CH, HC), jnp.bfloat16),
            pltpu.VMEM((2, CH, HC), jnp.bfloat16),
            pltpu.VMEM((2, CH, HC), jnp.bfloat16),
            pltpu.VMEM((2, CH, HC), jnp.bfloat16),
            pltpu.SemaphoreType.DMA((2,)),
            pltpu.SemaphoreType.DMA((2,)),
            pltpu.SemaphoreType.DMA((2,)),
            pltpu.SemaphoreType.DMA((2,)),
            pltpu.SemaphoreType.REGULAR,
            pltpu.SemaphoreType.REGULAR,
        ],
        compiler_params=pltpu.CompilerParams(
            collective_id=0, vmem_limit_bytes=64 * 1024 * 1024),
    )(xb, wb, sp)


# device time: 278641 ns/iter; 3.1697x vs baseline; 1.7558x over previous
import jax
import jax.numpy as jnp
from jax import lax
from jax.experimental import pallas as pl
from jax.experimental.pallas import tpu as pltpu

N_PLANE = 8
N_Z = 4
M = 4096
N_COL = 2048
HC = N_COL // 2
PCH = M // N_PLANE
SUB = PCH // N_Z


def kernel(x, w_mat, scale_x, scale_w):
    xb = x.astype(jnp.bfloat16)
    wb = w_mat.astype(jnp.bfloat16)
    sp = (scale_x.astype(jnp.float32) * scale_w.astype(jnp.float32)).reshape(1, 1)

    def body(x_ref, w_ref, sp_ref, out_ref,
             stageAR, stageAL, commAR, commAL,
             stageBR, stageBL, commBR, commBL,
             planeR, planeL, finalR, finalL,
             sendR, sendL, recvR, recvL,
             sendBR, sendBL, recvBR, recvBL,
             creditR, creditL, creditBR, creditBL):
        d = lax.axis_index("i")
        z = lax.div(d, N_PLANE)
        s_idx = lax.rem(d, N_PLANE)
        y_me = lax.div(s_idx, 2)
        x_me = lax.rem(s_idx + y_me, 2)
        c = jnp.where(x_me == 1, 1 + y_me, lax.rem(8 - y_me, 8))

        def plane_pos(cc):
            xx = jnp.where(cc == 0, 0, jnp.where(cc <= 4, 1, 0))
            yy = jnp.where(cc == 0, 0, jnp.where(cc <= 4, cc - 1, 8 - cc))
            ss = 2 * yy + lax.rem(xx + yy, 2)
            return z * N_PLANE + ss

        qr = plane_pos(lax.rem(c + 1, N_PLANE))
        ql = plane_pos(lax.rem(c + N_PLANE - 1, N_PLANE))
        zr = lax.rem(z + 1, N_Z) * N_PLANE + s_idx
        zl = lax.rem(z + N_Z - 1, N_Z) * N_PLANE + s_idx

        barrier = pltpu.get_barrier_semaphore()
        for nbr in (ql, qr):
            pl.semaphore_signal(barrier, inc=1, device_id=(nbr,),
                                device_id_type=pl.DeviceIdType.MESH)
        pl.semaphore_wait(barrier, 2)

        pl.semaphore_signal(creditBR, inc=2, device_id=(zl,),
                            device_id_type=pl.DeviceIdType.MESH)
        pl.semaphore_signal(creditBL, inc=2, device_id=(zr,),
                            device_id_type=pl.DeviceIdType.MESH)

        scale = sp_ref[0, 0]

        def pchunkA(j, dir_):
            xa = x_ref[pl.ds(j * PCH, PCH), :]
            wa = w_ref[:, dir_ * HC:(dir_ + 1) * HC]
            return lax.dot_general(
                xa, wa,
                dimension_numbers=(((1,), (0,)), ((), ())),
                preferred_element_type=jnp.float32)

        def silu(acc):
            yv = acc * scale
            return yv / (1.0 + jnp.exp(-jnp.clip(yv, -60.0, 60.0)))

        def exchange(stR, cmR, stL, cmL, ssR, rsR, ssL, rsL, slot, tR, tL):
            rr = pltpu.make_async_remote_copy(
                src_ref=stR.at[slot], dst_ref=cmR.at[slot],
                send_sem=ssR.at[slot], recv_sem=rsR.at[slot],
                device_id=(tR,), device_id_type=pl.DeviceIdType.MESH)
            rl = pltpu.make_async_remote_copy(
                src_ref=stL.at[slot], dst_ref=cmL.at[slot],
                send_sem=ssL.at[slot], recv_sem=rsL.at[slot],
                device_id=(tL,), device_id_type=pl.DeviceIdType.MESH)
            rr.start()
            rl.start()
            return rr, rl

        stageAR[0, :, :] = pchunkA(c, 0).astype(jnp.bfloat16)
        stageAL[0, :, :] = pchunkA(c, 1).astype(jnp.bfloat16)

        for s in range(N_PLANE - 1):
            slot, nxt = s % 2, (s + 1) % 2
            if s >= 2:
                pl.semaphore_wait(creditR, 1)
                pl.semaphore_wait(creditL, 1)
            rr, rl = exchange(stageAR, commAR, stageAL, commAL,
                              sendR, recvR, sendL, recvL, slot, qr, ql)
            jR = lax.rem(c - (s + 1) + 2 * N_PLANE, N_PLANE)
            jL = lax.rem(c + (s + 1), N_PLANE)
            pcR = pchunkA(jR, 0)
            pcL = pchunkA(jL, 1)
            rr.wait_send()
            rl.wait_send()
            rr.wait_recv()
            rl.wait_recv()
            accR = commAR[slot, :, :].astype(jnp.float32) + pcR
            accL = commAL[slot, :, :].astype(jnp.float32) + pcL
            if s < N_PLANE - 2:
                stageAR[nxt, :, :] = accR.astype(jnp.bfloat16)
                stageAL[nxt, :, :] = accL.astype(jnp.bfloat16)
            else:
                planeR[:, :] = accR.astype(jnp.bfloat16)
                planeL[:, :] = accL.astype(jnp.bfloat16)
            pl.semaphore_signal(creditR, inc=1, device_id=(ql,),
                                device_id_type=pl.DeviceIdType.MESH)
            pl.semaphore_signal(creditL, inc=1, device_id=(qr,),
                                device_id_type=pl.DeviceIdType.MESH)

        rows_R = lax.rem(c + 1, N_PLANE) * PCH
        rows_L = lax.rem(c + N_PLANE - 1, N_PLANE) * PCH

        stageBR[0, :, :] = planeR[pl.ds(z * SUB, SUB), :]
        stageBL[0, :, :] = planeL[pl.ds(z * SUB, SUB), :]

        for b in range(2 * N_Z - 2):
            slot, nxt = b % 2, (b + 1) % 2
            pl.semaphore_wait(creditBR, 1)
            pl.semaphore_wait(creditBL, 1)
            rr, rl = exchange(stageBR, commBR, stageBL, commBL,
                              sendBR, recvBR, sendBL, recvBL, slot, zr, zl)
            rr.wait_send()
            rl.wait_send()
            rr.wait_recv()
            rl.wait_recv()
            if b < N_Z - 1:
                mR = lax.rem(z - (b + 1) + 2 * N_Z, N_Z)
                mL = lax.rem(z + (b + 1), N_Z)
                accR = (commBR[slot, :, :].astype(jnp.float32)
                        + planeR[pl.ds(mR * SUB, SUB), :].astype(jnp.float32))
                accL = (commBL[slot, :, :].astype(jnp.float32)
                        + planeL[pl.ds(mL * SUB, SUB), :].astype(jnp.float32))
                if b == N_Z - 2:
                    yR = silu(accR)
                    yL = silu(accL)
                    out_ref[pl.ds(rows_R + mR * SUB, SUB), 0:HC] = yR
                    out_ref[pl.ds(rows_L + mL * SUB, SUB), HC:N_COL] = yL
                    finalR[pl.ds(mR * SUB, SUB), :] = yR.astype(jnp.bfloat16)
                    finalL[pl.ds(mL * SUB, SUB), :] = yL.astype(jnp.bfloat16)
                    stageBR[nxt, :, :] = yR.astype(jnp.bfloat16)
                    stageBL[nxt, :, :] = yL.astype(jnp.bfloat16)
                else:
                    stageBR[nxt, :, :] = accR.astype(jnp.bfloat16)
                    stageBL[nxt, :, :] = accL.astype(jnp.bfloat16)
            else:
                t = b - (N_Z - 1)
                mR = lax.rem(z - t + 2 * N_Z, N_Z)
                mL = lax.rem(z + t, N_Z)
                finalR[pl.ds(mR * SUB, SUB), :] = commBR[slot, :, :]
                finalL[pl.ds(mL * SUB, SUB), :] = commBL[slot, :, :]
                out_ref[pl.ds(rows_R + mR * SUB, SUB), 0:HC] = (
                    commBR[slot, :, :].astype(jnp.float32))
                out_ref[pl.ds(rows_L + mL * SUB, SUB), HC:N_COL] = (
                    commBL[slot, :, :].astype(jnp.float32))
                if b < 2 * N_Z - 3:
                    stageBR[nxt, :, :] = commBR[slot, :, :]
                    stageBL[nxt, :, :] = commBL[slot, :, :]
            if b <= 2 * N_Z - 5:
                pl.semaphore_signal(creditBR, inc=1, device_id=(zl,),
                                    device_id_type=pl.DeviceIdType.MESH)
                pl.semaphore_signal(creditBL, inc=1, device_id=(zr,),
                                    device_id_type=pl.DeviceIdType.MESH)

        stageAR[0, :, :] = finalR[:, :]
        stageAL[0, :, :] = finalL[:, :]

        for t in range(N_PLANE - 1):
            slot, nxt = t % 2, (t + 1) % 2
            if t == 0:
                pl.semaphore_wait(creditR, 2)
                pl.semaphore_wait(creditL, 2)
            elif t >= 2:
                pl.semaphore_wait(creditR, 1)
                pl.semaphore_wait(creditL, 1)
            rr, rl = exchange(stageAR, commAR, stageAL, commAL,
                              sendR, recvR, sendL, recvL, slot, qr, ql)
            rr.wait_send()
            rl.wait_send()
            rr.wait_recv()
            rl.wait_recv()
            rowsRr = lax.rem(c - t + 2 * N_PLANE, N_PLANE) * PCH
            rowsLr = lax.rem(c + t, N_PLANE) * PCH
            out_ref[pl.ds(rowsRr, PCH), 0:HC] = (
                commAR[slot, :, :].astype(jnp.float32))
            out_ref[pl.ds(rowsLr, PCH), HC:N_COL] = (
                commAL[slot, :, :].astype(jnp.float32))
            if t < N_PLANE - 2:
                stageAR[nxt, :, :] = commAR[slot, :, :]
                stageAL[nxt, :, :] = commAL[slot, :, :]
            if t <= N_PLANE - 4:
                pl.semaphore_signal(creditR, inc=1, device_id=(ql,),
                                    device_id_type=pl.DeviceIdType.MESH)
                pl.semaphore_signal(creditL, inc=1, device_id=(qr,),
                                    device_id_type=pl.DeviceIdType.MESH)

    return pl.pallas_call(
        body,
        out_shape=jax.ShapeDtypeStruct((M, N_COL), jnp.float32),
        in_specs=[
            pl.BlockSpec(memory_space=pltpu.VMEM),
            pl.BlockSpec(memory_space=pltpu.VMEM),
            pl.BlockSpec(memory_space=pltpu.SMEM),
        ],
        out_specs=pl.BlockSpec(memory_space=pltpu.VMEM),
        scratch_shapes=[
            pltpu.VMEM((2, PCH, HC), jnp.bfloat16),
            pltpu.VMEM((2, PCH, HC), jnp.bfloat16),
            pltpu.VMEM((2, PCH, HC), jnp.bfloat16),
            pltpu.VMEM((2, PCH, HC), jnp.bfloat16),
            pltpu.VMEM((2, SUB, HC), jnp.bfloat16),
            pltpu.VMEM((2, SUB, HC), jnp.bfloat16),
            pltpu.VMEM((2, SUB, HC), jnp.bfloat16),
            pltpu.VMEM((2, SUB, HC), jnp.bfloat16),
            pltpu.VMEM((PCH, HC), jnp.bfloat16),
            pltpu.VMEM((PCH, HC), jnp.bfloat16),
            pltpu.VMEM((PCH, HC), jnp.bfloat16),
            pltpu.VMEM((PCH, HC), jnp.bfloat16),
            pltpu.SemaphoreType.DMA((2,)),
            pltpu.SemaphoreType.DMA((2,)),
            pltpu.SemaphoreType.DMA((2,)),
            pltpu.SemaphoreType.DMA((2,)),
            pltpu.SemaphoreType.DMA((2,)),
            pltpu.SemaphoreType.DMA((2,)),
            pltpu.SemaphoreType.DMA((2,)),
            pltpu.SemaphoreType.DMA((2,)),
            pltpu.SemaphoreType.REGULAR,
            pltpu.SemaphoreType.REGULAR,
            pltpu.SemaphoreType.REGULAR,
            pltpu.SemaphoreType.REGULAR,
        ],
        compiler_params=pltpu.CompilerParams(
            collective_id=0, vmem_limit_bytes=64 * 1024 * 1024),
    )(xb, wb, sp)
